# baseline (speedup 1.0000x reference)
import functools

import jax
import jax.numpy as jnp
from jax import lax
from jax.experimental import pallas as pl
from jax.experimental.pallas import tpu as pltpu
from jax.experimental.pallas import tpu_sc as plsc

_K = 2
_LOAD = 1.25


def _router_body(cap, dump_row, x_ref, w_ref, b_ref, dst_ref, gcl_ref, prob_ref):
    b = pl.program_id(0)
    T = x_ref.shape[1]
    E = w_ref.shape[1]

    x = x_ref[0]
    logits = jnp.dot(x, w_ref[...], preferred_element_type=jnp.float32)
    logits = logits + b_ref[...]
    m = jnp.max(logits, axis=1, keepdims=True)
    p = jnp.exp(logits - m)
    p = p / jnp.sum(p, axis=1, keepdims=True)

    e_iota = lax.broadcasted_iota(jnp.int32, (T, E), 1)
    m0 = jnp.max(p, axis=1, keepdims=True)
    i0 = jnp.min(jnp.where(p == m0, e_iota, E), axis=1)
    pm = jnp.where(e_iota == i0[:, None], -1.0, p)
    m1 = jnp.max(pm, axis=1, keepdims=True)
    i1 = jnp.min(jnp.where(pm == m1, e_iota, E), axis=1)

    oh0 = (e_iota == i0[:, None]).astype(jnp.bfloat16)
    oh1 = (e_iota == i1[:, None]).astype(jnp.bfloat16)
    r_iota = lax.broadcasted_iota(jnp.int32, (T, T), 0)
    c_iota = lax.broadcasted_iota(jnp.int32, (T, T), 1)
    L = (c_iota <= r_iota).astype(jnp.bfloat16)
    cnt = jnp.dot(L, jnp.concatenate([oh0, oh1], axis=1),
                  preferred_element_type=jnp.float32)
    cnt0 = cnt[:, :E]
    cnt1 = cnt[:, E:] + cnt0[T - 1:T, :]
    oh0f = oh0.astype(jnp.float32)
    oh1f = oh1.astype(jnp.float32)
    pos0 = jnp.sum(cnt0 * oh0f, axis=1).astype(jnp.int32) - 1
    pos1 = jnp.sum(cnt1 * oh1f, axis=1).astype(jnp.int32) - 1

    def finalize(idxk, pos):
        rowbase = idxk * (cap * 2) + b * cap
        dst = jnp.where(pos < cap, rowbase + pos, dump_row)
        gcl = rowbase + jnp.minimum(pos, cap - 1)
        return dst.reshape(1, 1, 1, T), gcl.reshape(1, 1, 1, T)

    d0, c0 = finalize(i0, pos0)
    d1, c1 = finalize(i1, pos1)
    dst_ref[...] = jnp.concatenate([d0, d1], axis=1)
    gcl_ref[...] = jnp.concatenate([c0, c1], axis=1)
    prob_ref[...] = jnp.concatenate([m0[:, 0].reshape(1, 1, 1, T),
                                     m1[:, 0].reshape(1, 1, 1, T)], axis=1)


def _run_router(x, router_w, router_b, cap, dump_row):
    B, T, C = x.shape
    E = router_w.shape[1]
    out4 = jax.ShapeDtypeStruct((B, _K, 1, T), jnp.int32)
    outp = jax.ShapeDtypeStruct((B, _K, 1, T), jnp.float32)
    return pl.pallas_call(
        functools.partial(_router_body, cap, dump_row),
        grid=(B,),
        in_specs=[
            pl.BlockSpec((1, T, C), lambda b: (b, 0, 0)),
            pl.BlockSpec((C, E), lambda b: (0, 0)),
            pl.BlockSpec((1, E), lambda b: (0, 0)),
        ],
        out_specs=[
            pl.BlockSpec((1, _K, 1, T), lambda b: (b, 0, 0, 0)),
            pl.BlockSpec((1, _K, 1, T), lambda b: (b, 0, 0, 0)),
            pl.BlockSpec((1, _K, 1, T), lambda b: (b, 0, 0, 0)),
        ],
        out_shape=[out4, out4, outp],
    )(x, router_w, router_b.reshape(1, E))


def _make_dispatch(T, n_slots, n_rows_pad, C, chunk):
    n_chunks = n_slots // chunk
    cph = T // chunk
    mesh = plsc.VectorSubcoreMesh(core_axis_name="c", subcore_axis_name="s")
    nw = mesh.num_cores * mesh.num_subcores
    per_w = n_chunks // nw

    @functools.partial(
        pl.kernel,
        out_type=jax.ShapeDtypeStruct((n_rows_pad, C), jnp.float32),
        mesh=mesh,
        scratch_types=[
            pltpu.VMEM((2, chunk), jnp.int32),
            pltpu.VMEM((chunk, C), jnp.float32),
            pltpu.VMEM((chunk, C), jnp.float32),
            pltpu.SemaphoreType.DMA,
            pltpu.SemaphoreType.DMA,
            pltpu.SemaphoreType.DMA,
            pltpu.SemaphoreType.DMA,
        ],
    )
    def dispatch(x_hbm, dst_hbm, ei_hbm, idx_v, rows0, rows1, ld0, ld1, sc0, sc1):
        wid = lax.axis_index("s") * mesh.num_cores + lax.axis_index("c")
        rows = (rows0, rows1)
        ldsem = (ld0, ld1)
        scsem = (sc0, sc1)
        hld = [None, None]
        hsc = [None, None]
        for j in range(per_w + 1):
            cur = j % 2
            if j < per_w:
                g = wid * per_w + j
                b = g // (_K * cph)
                xbase = (b * T + (g % cph) * chunk).astype(jnp.int32)
                if hsc[cur] is not None:
                    hsc[cur].wait()
                pltpu.sync_copy(dst_hbm.at[pl.ds(g * chunk, chunk)],
                                idx_v.at[cur])
                hld[cur] = pltpu.async_copy(x_hbm.at[pl.ds(xbase, chunk)],
                                            rows[cur], ldsem[cur])
            if j >= 1:
                prev = (j - 1) % 2
                hld[prev].wait()
                hsc[prev] = pltpu.async_copy(rows[prev],
                                             ei_hbm.at[idx_v.at[prev]],
                                             scsem[prev])
        for h in hsc:
            if h is not None:
                h.wait()

    return dispatch


def _ffn_body(x_ref, w1_ref, b1_ref, wg_ref, bg_ref, w2_ref, b2_ref, out_ref):
    hb = pl.program_id(1)
    x = x_ref[...]
    h = jnp.dot(x, w1_ref[0], preferred_element_type=jnp.float32) + b1_ref[0]
    g = jnp.dot(x, wg_ref[0], preferred_element_type=jnp.float32) + bg_ref[0]
    hg = h * g
    s = hg * jax.nn.sigmoid(hg)
    acc = jnp.dot(s, w2_ref[0], preferred_element_type=jnp.float32)

    @pl.when(hb == 0)
    def _():
        out_ref[...] = jnp.broadcast_to(b2_ref[0], out_ref.shape)
    out_ref[...] += acc


def _run_ffn(ei, w1, b1, wg, bg, w2, b2, n_rows, hblk):
    E, C, H = w1.shape
    rows = n_rows // E
    grid = (E, H // hblk)
    return pl.pallas_call(
        _ffn_body,
        grid=grid,
        in_specs=[
            pl.BlockSpec((rows, C), lambda e, h: (e, 0)),
            pl.BlockSpec((1, C, hblk), lambda e, h: (e, 0, h)),
            pl.BlockSpec((1, 1, hblk), lambda e, h: (e, 0, h)),
            pl.BlockSpec((1, C, hblk), lambda e, h: (e, 0, h)),
            pl.BlockSpec((1, 1, hblk), lambda e, h: (e, 0, h)),
            pl.BlockSpec((1, hblk, C), lambda e, h: (e, h, 0)),
            pl.BlockSpec((1, 1, C), lambda e, h: (e, 0, 0)),
        ],
        out_specs=pl.BlockSpec((rows, C), lambda e, h: (e, 0)),
        out_shape=jax.ShapeDtypeStruct((n_rows, C), jnp.float32),
    )(ei, w1, b1, wg, bg, w2, b2)


def _make_gather(T, n_tok, C, chunk):
    n_chunks = n_tok // chunk
    cpb = T // chunk
    mesh = plsc.VectorSubcoreMesh(core_axis_name="c", subcore_axis_name="s")
    nw = mesh.num_cores * mesh.num_subcores
    per_w = n_chunks // nw

    n_jobs = per_w * _K

    @functools.partial(
        pl.kernel,
        out_type=[
            jax.ShapeDtypeStruct((n_tok, C), jnp.float32),
            jax.ShapeDtypeStruct((n_tok, C), jnp.float32),
        ],
        mesh=mesh,
        scratch_types=[
            pltpu.VMEM((2, chunk), jnp.int32),
            pltpu.VMEM((chunk, C), jnp.float32),
            pltpu.VMEM((chunk, C), jnp.float32),
            pltpu.SemaphoreType.DMA,
            pltpu.SemaphoreType.DMA,
            pltpu.SemaphoreType.DMA,
            pltpu.SemaphoreType.DMA,
        ],
    )
    def gather(eo_hbm, gcl_hbm, out0_hbm, out1_hbm,
               idx_v, rows0, rows1, g0, g1, s0, s1):
        wid = lax.axis_index("s") * mesh.num_cores + lax.axis_index("c")
        rows = (rows0, rows1)
        gsem = (g0, g1)
        ssem = (s0, s1)
        outs = (out0_hbm, out1_hbm)
        hg = [None, None]
        hs = [None, None]
        jobs = []
        for i in range(per_w):
            for k in range(_K):
                jobs.append((i, k))
        for j in range(n_jobs + 1):
            cur = j % 2
            if j < n_jobs:
                i, k = jobs[j]
                g = wid * per_w + i
                b = g // cpb
                goff = (b * _K * T + k * T + (g % cpb) * chunk).astype(jnp.int32)
                if hs[cur] is not None:
                    hs[cur].wait()
                pltpu.sync_copy(gcl_hbm.at[pl.ds(goff, chunk)], idx_v.at[cur])
                hg[cur] = pltpu.async_copy(eo_hbm.at[idx_v.at[cur]],
                                           rows[cur], gsem[cur])
            if j >= 1:
                prev = (j - 1) % 2
                pi, pk = jobs[j - 1]
                pbase = (wid * per_w + pi) * chunk
                hg[prev].wait()
                hs[prev] = pltpu.async_copy(rows[prev],
                                            outs[pk].at[pl.ds(pbase, chunk)],
                                            ssem[prev])
        for h in hs:
            if h is not None:
                h.wait()

    return gather


def _combine_body(g0_ref, g1_ref, p0_ref, p1_ref, y_ref):
    y_ref[...] = p0_ref[...] * g0_ref[...] + p1_ref[...] * g1_ref[...]


def _run_combine(g0, g1, probs_flat, T, blk):
    n, C = g0.shape
    cpb = T // blk
    return pl.pallas_call(
        _combine_body,
        grid=(n // blk,),
        in_specs=[
            pl.BlockSpec((blk, C), lambda i: (i, 0)),
            pl.BlockSpec((blk, C), lambda i: (i, 0)),
            pl.BlockSpec((blk, 1), lambda i: (i + (i // cpb) * cpb, 0)),
            pl.BlockSpec((blk, 1), lambda i: (i + (i // cpb) * cpb + cpb, 0)),
        ],
        out_specs=pl.BlockSpec((blk, C), lambda i: (i, 0)),
        out_shape=jax.ShapeDtypeStruct((n, C), jnp.float32),
    )(g0, g1, probs_flat, probs_flat)


def kernel(x, router_w, router_b, w_c_fc, b_c_fc, w_gate, b_gate, w_c_proj, b_c_proj):
    B, T, C = x.shape
    E = router_w.shape[1]
    cap = int(_LOAD * _K * T // E)
    n_rows = E * B * cap
    n_rows_pad = n_rows + 8
    dump_row = n_rows

    dst, gcl, probs = _run_router(x, router_w, router_b, cap, dump_row)

    ei = _make_dispatch(T, B * _K * T, n_rows_pad, C, 64)(x.reshape(-1, C),
                                                          dst.reshape(-1))

    hblk = 768 if (w_c_fc.shape[2] % 768 == 0) else w_c_fc.shape[2]
    eo = _run_ffn(ei, w_c_fc, b_c_fc, w_gate, b_gate, w_c_proj, b_c_proj,
                  n_rows, hblk)

    r0, r1 = _make_gather(T, B * T, C, 64)(eo, gcl.reshape(-1))

    y = _run_combine(r0, r1, probs.reshape(-1, 1), T, min(512, T))
    return y.reshape(B, T, C)


# speedup vs baseline: 1.0337x; 1.0337x over previous
import functools

import jax
import jax.numpy as jnp
from jax import lax
from jax.experimental import pallas as pl
from jax.experimental.pallas import tpu as pltpu
from jax.experimental.pallas import tpu_sc as plsc

_K = 2
_LOAD = 1.25


def _router_body(cap, dump_row, x_ref, w_ref, b_ref, dst_ref, gcl_ref, prob_ref):
    b = pl.program_id(0)
    T = x_ref.shape[1]
    E = w_ref.shape[1]

    x = x_ref[0]
    logits = jnp.dot(x, w_ref[...], preferred_element_type=jnp.float32)
    logits = logits + b_ref[...]
    m = jnp.max(logits, axis=1, keepdims=True)
    p = jnp.exp(logits - m)
    p = p / jnp.sum(p, axis=1, keepdims=True)

    e_iota = lax.broadcasted_iota(jnp.int32, (T, E), 1)
    m0 = jnp.max(p, axis=1, keepdims=True)
    i0 = jnp.min(jnp.where(p == m0, e_iota, E), axis=1)
    pm = jnp.where(e_iota == i0[:, None], -1.0, p)
    m1 = jnp.max(pm, axis=1, keepdims=True)
    i1 = jnp.min(jnp.where(pm == m1, e_iota, E), axis=1)

    oh0 = (e_iota == i0[:, None]).astype(jnp.bfloat16)
    oh1 = (e_iota == i1[:, None]).astype(jnp.bfloat16)
    r_iota = lax.broadcasted_iota(jnp.int32, (T, T), 0)
    c_iota = lax.broadcasted_iota(jnp.int32, (T, T), 1)
    L = (c_iota <= r_iota).astype(jnp.bfloat16)
    cnt = jnp.dot(L, jnp.concatenate([oh0, oh1], axis=1),
                  preferred_element_type=jnp.float32)
    cnt0 = cnt[:, :E]
    cnt1 = cnt[:, E:] + cnt0[T - 1:T, :]
    oh0f = oh0.astype(jnp.float32)
    oh1f = oh1.astype(jnp.float32)
    pos0 = jnp.sum(cnt0 * oh0f, axis=1).astype(jnp.int32) - 1
    pos1 = jnp.sum(cnt1 * oh1f, axis=1).astype(jnp.int32) - 1

    def finalize(idxk, pos):
        rowbase = idxk * (cap * 2) + b * cap
        dst = jnp.where(pos < cap, rowbase + pos, dump_row)
        gcl = rowbase + jnp.minimum(pos, cap - 1)
        return dst.reshape(1, 1, 1, T), gcl.reshape(1, 1, 1, T)

    d0, c0 = finalize(i0, pos0)
    d1, c1 = finalize(i1, pos1)
    dst_ref[...] = jnp.concatenate([d0, d1], axis=1)
    gcl_ref[...] = jnp.concatenate([c0, c1], axis=1)
    prob_ref[...] = jnp.concatenate([m0[:, 0].reshape(1, 1, 1, T),
                                     m1[:, 0].reshape(1, 1, 1, T)], axis=1)


def _run_router(x, router_w, router_b, cap, dump_row):
    B, T, C = x.shape
    E = router_w.shape[1]
    out4 = jax.ShapeDtypeStruct((B, _K, 1, T), jnp.int32)
    outp = jax.ShapeDtypeStruct((B, _K, 1, T), jnp.float32)
    return pl.pallas_call(
        functools.partial(_router_body, cap, dump_row),
        grid=(B,),
        in_specs=[
            pl.BlockSpec((1, T, C), lambda b: (b, 0, 0)),
            pl.BlockSpec((C, E), lambda b: (0, 0)),
            pl.BlockSpec((1, E), lambda b: (0, 0)),
        ],
        out_specs=[
            pl.BlockSpec((1, _K, 1, T), lambda b: (b, 0, 0, 0)),
            pl.BlockSpec((1, _K, 1, T), lambda b: (b, 0, 0, 0)),
            pl.BlockSpec((1, _K, 1, T), lambda b: (b, 0, 0, 0)),
        ],
        out_shape=[out4, out4, outp],
    )(x, router_w, router_b.reshape(1, E))


def _make_dispatch(T, n_slots, n_rows_pad, C, chunk):
    n_chunks = n_slots // chunk
    cph = T // chunk
    mesh = plsc.VectorSubcoreMesh(core_axis_name="c", subcore_axis_name="s")
    nw = mesh.num_cores * mesh.num_subcores
    per_w = n_chunks // nw

    @functools.partial(
        pl.kernel,
        out_type=jax.ShapeDtypeStruct((n_rows_pad, C), jnp.float32),
        mesh=mesh,
        scratch_types=[
            pltpu.VMEM((2, chunk), jnp.int32),
            pltpu.VMEM((chunk, C), jnp.float32),
            pltpu.VMEM((chunk, C), jnp.float32),
            pltpu.SemaphoreType.DMA,
            pltpu.SemaphoreType.DMA,
            pltpu.SemaphoreType.DMA,
            pltpu.SemaphoreType.DMA,
        ],
    )
    def dispatch(x_hbm, dst_hbm, ei_hbm, idx_v, rows0, rows1, ld0, ld1, sc0, sc1):
        wid = lax.axis_index("s") * mesh.num_cores + lax.axis_index("c")
        rows = (rows0, rows1)
        ldsem = (ld0, ld1)
        scsem = (sc0, sc1)
        hld = [None, None]
        hsc = [None, None]
        for j in range(per_w + 1):
            cur = j % 2
            if j < per_w:
                g = wid * per_w + j
                b = g // (_K * cph)
                xbase = (b * T + (g % cph) * chunk).astype(jnp.int32)
                if hsc[cur] is not None:
                    hsc[cur].wait()
                pltpu.sync_copy(dst_hbm.at[pl.ds(g * chunk, chunk)],
                                idx_v.at[cur])
                hld[cur] = pltpu.async_copy(x_hbm.at[pl.ds(xbase, chunk)],
                                            rows[cur], ldsem[cur])
            if j >= 1:
                prev = (j - 1) % 2
                hld[prev].wait()
                hsc[prev] = pltpu.async_copy(rows[prev],
                                             ei_hbm.at[idx_v.at[prev]],
                                             scsem[prev])
        for h in hsc:
            if h is not None:
                h.wait()

    return dispatch


def _ffn_body(x_ref, w1_ref, b1_ref, wg_ref, bg_ref, w2_ref, b2_ref, out_ref):
    hb = pl.program_id(1)
    x = x_ref[...]
    h = jnp.dot(x, w1_ref[0], preferred_element_type=jnp.float32) + b1_ref[0]
    g = jnp.dot(x, wg_ref[0], preferred_element_type=jnp.float32) + bg_ref[0]
    hg = h * g
    s = hg * jax.nn.sigmoid(hg)
    acc = jnp.dot(s, w2_ref[0], preferred_element_type=jnp.float32)

    @pl.when(hb == 0)
    def _():
        out_ref[...] = jnp.broadcast_to(b2_ref[0], out_ref.shape)
    out_ref[...] += acc


def _run_ffn(ei, w1, b1, wg, bg, w2, b2, n_rows, hblk):
    E, C, H = w1.shape
    rows = n_rows // E
    grid = (E, H // hblk)
    return pl.pallas_call(
        _ffn_body,
        grid=grid,
        in_specs=[
            pl.BlockSpec((rows, C), lambda e, h: (e, 0)),
            pl.BlockSpec((1, C, hblk), lambda e, h: (e, 0, h)),
            pl.BlockSpec((1, 1, hblk), lambda e, h: (e, 0, h)),
            pl.BlockSpec((1, C, hblk), lambda e, h: (e, 0, h)),
            pl.BlockSpec((1, 1, hblk), lambda e, h: (e, 0, h)),
            pl.BlockSpec((1, hblk, C), lambda e, h: (e, h, 0)),
            pl.BlockSpec((1, 1, C), lambda e, h: (e, 0, 0)),
        ],
        out_specs=pl.BlockSpec((rows, C), lambda e, h: (e, 0)),
        out_shape=jax.ShapeDtypeStruct((n_rows, C), jnp.float32),
    )(ei, w1, b1, wg, bg, w2, b2)


def _make_gather(T, n_tok, C, chunk):
    n_chunks = n_tok // chunk
    cpb = T // chunk
    mesh = plsc.VectorSubcoreMesh(core_axis_name="c", subcore_axis_name="s")
    nw = mesh.num_cores * mesh.num_subcores
    per_w = n_chunks // nw

    n_jobs = per_w * _K

    @functools.partial(
        pl.kernel,
        out_type=[
            jax.ShapeDtypeStruct((n_tok, C), jnp.float32),
            jax.ShapeDtypeStruct((n_tok, C), jnp.float32),
        ],
        mesh=mesh,
        scratch_types=[
            pltpu.VMEM((2, chunk), jnp.int32),
            pltpu.VMEM((chunk, C), jnp.float32),
            pltpu.VMEM((chunk, C), jnp.float32),
            pltpu.SemaphoreType.DMA,
            pltpu.SemaphoreType.DMA,
            pltpu.SemaphoreType.DMA,
            pltpu.SemaphoreType.DMA,
        ],
    )
    def gather(eo_hbm, gcl_hbm, out0_hbm, out1_hbm,
               idx_v, rows0, rows1, g0, g1, s0, s1):
        wid = lax.axis_index("s") * mesh.num_cores + lax.axis_index("c")
        rows = (rows0, rows1)
        gsem = (g0, g1)
        ssem = (s0, s1)
        outs = (out0_hbm, out1_hbm)
        hg = [None, None]
        hs = [None, None]
        jobs = []
        for i in range(per_w):
            for k in range(_K):
                jobs.append((i, k))
        for j in range(n_jobs + 1):
            cur = j % 2
            if j < n_jobs:
                i, k = jobs[j]
                g = wid * per_w + i
                b = g // cpb
                goff = (b * _K * T + k * T + (g % cpb) * chunk).astype(jnp.int32)
                if hs[cur] is not None:
                    hs[cur].wait()
                pltpu.sync_copy(gcl_hbm.at[pl.ds(goff, chunk)], idx_v.at[cur])
                hg[cur] = pltpu.async_copy(eo_hbm.at[idx_v.at[cur]],
                                           rows[cur], gsem[cur])
            if j >= 1:
                prev = (j - 1) % 2
                pi, pk = jobs[j - 1]
                pbase = (wid * per_w + pi) * chunk
                hg[prev].wait()
                hs[prev] = pltpu.async_copy(rows[prev],
                                            outs[pk].at[pl.ds(pbase, chunk)],
                                            ssem[prev])
        for h in hs:
            if h is not None:
                h.wait()

    return gather


def _combine_body(g0_ref, g1_ref, p0_ref, p1_ref, y_ref):
    y_ref[...] = p0_ref[...] * g0_ref[...] + p1_ref[...] * g1_ref[...]


def _run_combine(g0, g1, probs_flat, T, blk):
    n, C = g0.shape
    cpb = T // blk
    return pl.pallas_call(
        _combine_body,
        grid=(n // blk,),
        in_specs=[
            pl.BlockSpec((blk, C), lambda i: (i, 0)),
            pl.BlockSpec((blk, C), lambda i: (i, 0)),
            pl.BlockSpec((blk, 1), lambda i: (i + (i // cpb) * cpb, 0)),
            pl.BlockSpec((blk, 1), lambda i: (i + (i // cpb) * cpb + cpb, 0)),
        ],
        out_specs=pl.BlockSpec((blk, C), lambda i: (i, 0)),
        out_shape=jax.ShapeDtypeStruct((n, C), jnp.float32),
    )(g0, g1, probs_flat, probs_flat)


def kernel(x, router_w, router_b, w_c_fc, b_c_fc, w_gate, b_gate, w_c_proj, b_c_proj):
    B, T, C = x.shape
    E = router_w.shape[1]
    cap = int(_LOAD * _K * T // E)
    n_rows = E * B * cap
    n_rows_pad = n_rows + 8
    dump_row = n_rows

    dst, gcl, probs = _run_router(x, router_w, router_b, cap, dump_row)

    ei = _make_dispatch(T, B * _K * T, n_rows_pad, C, 64)(x.reshape(-1, C),
                                                          dst.reshape(-1))

    hblk = 1024 if (w_c_fc.shape[2] % 1024 == 0) else w_c_fc.shape[2]
    eo = _run_ffn(ei, w_c_fc, b_c_fc, w_gate, b_gate, w_c_proj, b_c_proj,
                  n_rows, hblk)

    r0, r1 = _make_gather(T, B * T, C, 64)(eo, gcl.reshape(-1))

    y = _run_combine(r0, r1, probs.reshape(-1, 1), T, min(512, T))
    return y.reshape(B, T, C)


# speedup vs baseline: 1.0514x; 1.0171x over previous
import functools

import jax
import jax.numpy as jnp
from jax import lax
from jax.experimental import pallas as pl
from jax.experimental.pallas import tpu as pltpu
from jax.experimental.pallas import tpu_sc as plsc

_K = 2
_LOAD = 1.25


def _router_body(cap, dump_row, x_ref, w_ref, b_ref, dst_ref, gcl_ref, prob_ref):
    b = pl.program_id(0)
    T = x_ref.shape[1]
    E = w_ref.shape[1]

    x = x_ref[0]
    logits = jnp.dot(x, w_ref[...], preferred_element_type=jnp.float32)
    logits = logits + b_ref[...]
    m = jnp.max(logits, axis=1, keepdims=True)
    p = jnp.exp(logits - m)
    p = p / jnp.sum(p, axis=1, keepdims=True)

    e_iota = lax.broadcasted_iota(jnp.int32, (T, E), 1)
    m0 = jnp.max(p, axis=1, keepdims=True)
    i0 = jnp.min(jnp.where(p == m0, e_iota, E), axis=1)
    pm = jnp.where(e_iota == i0[:, None], -1.0, p)
    m1 = jnp.max(pm, axis=1, keepdims=True)
    i1 = jnp.min(jnp.where(pm == m1, e_iota, E), axis=1)

    oh0 = (e_iota == i0[:, None]).astype(jnp.bfloat16)
    oh1 = (e_iota == i1[:, None]).astype(jnp.bfloat16)
    r_iota = lax.broadcasted_iota(jnp.int32, (T, T), 0)
    c_iota = lax.broadcasted_iota(jnp.int32, (T, T), 1)
    L = (c_iota <= r_iota).astype(jnp.bfloat16)
    cnt = jnp.dot(L, jnp.concatenate([oh0, oh1], axis=1),
                  preferred_element_type=jnp.float32)
    cnt0 = cnt[:, :E]
    cnt1 = cnt[:, E:] + cnt0[T - 1:T, :]
    oh0f = oh0.astype(jnp.float32)
    oh1f = oh1.astype(jnp.float32)
    pos0 = jnp.sum(cnt0 * oh0f, axis=1).astype(jnp.int32) - 1
    pos1 = jnp.sum(cnt1 * oh1f, axis=1).astype(jnp.int32) - 1

    def finalize(idxk, pos):
        rowbase = idxk * (cap * 2) + b * cap
        dst = jnp.where(pos < cap, rowbase + pos, dump_row)
        gcl = rowbase + jnp.minimum(pos, cap - 1)
        return dst.reshape(1, 1, 1, T), gcl.reshape(1, 1, 1, T)

    d0, c0 = finalize(i0, pos0)
    d1, c1 = finalize(i1, pos1)
    dst_ref[...] = jnp.concatenate([d0, d1], axis=1)
    gcl_ref[...] = jnp.concatenate([c0, c1], axis=1)
    prob_ref[...] = jnp.concatenate([m0[:, 0].reshape(1, 1, 1, T),
                                     m1[:, 0].reshape(1, 1, 1, T)], axis=1)


def _run_router(x, router_w, router_b, cap, dump_row):
    B, T, C = x.shape
    E = router_w.shape[1]
    out4 = jax.ShapeDtypeStruct((B, _K, 1, T), jnp.int32)
    outp = jax.ShapeDtypeStruct((B, _K, 1, T), jnp.float32)
    return pl.pallas_call(
        functools.partial(_router_body, cap, dump_row),
        grid=(B,),
        in_specs=[
            pl.BlockSpec((1, T, C), lambda b: (b, 0, 0)),
            pl.BlockSpec((C, E), lambda b: (0, 0)),
            pl.BlockSpec((1, E), lambda b: (0, 0)),
        ],
        out_specs=[
            pl.BlockSpec((1, _K, 1, T), lambda b: (b, 0, 0, 0)),
            pl.BlockSpec((1, _K, 1, T), lambda b: (b, 0, 0, 0)),
            pl.BlockSpec((1, _K, 1, T), lambda b: (b, 0, 0, 0)),
        ],
        out_shape=[out4, out4, outp],
    )(x, router_w, router_b.reshape(1, E))


def _make_dispatch(T, n_slots, n_rows_pad, C, chunk):
    n_chunks = n_slots // chunk
    cph = T // chunk
    mesh = plsc.VectorSubcoreMesh(core_axis_name="c", subcore_axis_name="s")
    nw = mesh.num_cores * mesh.num_subcores
    per_w = n_chunks // nw

    @functools.partial(
        pl.kernel,
        out_type=jax.ShapeDtypeStruct((n_rows_pad, C), jnp.float32),
        mesh=mesh,
        scratch_types=[
            pltpu.VMEM((2, chunk), jnp.int32),
            pltpu.VMEM((chunk, C), jnp.float32),
            pltpu.VMEM((chunk, C), jnp.float32),
            pltpu.SemaphoreType.DMA,
            pltpu.SemaphoreType.DMA,
            pltpu.SemaphoreType.DMA,
            pltpu.SemaphoreType.DMA,
        ],
    )
    def dispatch(x_hbm, dst_hbm, ei_hbm, idx_v, rows0, rows1, ld0, ld1, sc0, sc1):
        wid = lax.axis_index("s") * mesh.num_cores + lax.axis_index("c")
        rows = (rows0, rows1)
        ldsem = (ld0, ld1)
        scsem = (sc0, sc1)
        hld = [None, None]
        hsc = [None, None]
        for j in range(per_w + 1):
            cur = j % 2
            if j < per_w:
                g = wid * per_w + j
                b = g // (_K * cph)
                xbase = (b * T + (g % cph) * chunk).astype(jnp.int32)
                if hsc[cur] is not None:
                    hsc[cur].wait()
                pltpu.sync_copy(dst_hbm.at[pl.ds(g * chunk, chunk)],
                                idx_v.at[cur])
                hld[cur] = pltpu.async_copy(x_hbm.at[pl.ds(xbase, chunk)],
                                            rows[cur], ldsem[cur])
            if j >= 1:
                prev = (j - 1) % 2
                hld[prev].wait()
                hsc[prev] = pltpu.async_copy(rows[prev],
                                             ei_hbm.at[idx_v.at[prev]],
                                             scsem[prev])
        for h in hsc:
            if h is not None:
                h.wait()

    return dispatch


def _ffn_body(x_ref, w1_ref, b1_ref, wg_ref, bg_ref, w2_ref, b2_ref, out_ref):
    hb = pl.program_id(1)
    x = x_ref[...]
    h = jnp.dot(x, w1_ref[0], preferred_element_type=jnp.float32) + b1_ref[0]
    g = jnp.dot(x, wg_ref[0], preferred_element_type=jnp.float32) + bg_ref[0]
    hg = h * g
    s = hg * jax.nn.sigmoid(hg)
    acc = jnp.dot(s, w2_ref[0], preferred_element_type=jnp.float32)

    @pl.when(hb == 0)
    def _():
        out_ref[...] = jnp.broadcast_to(b2_ref[0], out_ref.shape)
    out_ref[...] += acc


def _run_ffn(ei, w1, b1, wg, bg, w2, b2, n_rows, hblk):
    E, C, H = w1.shape
    rows = n_rows // E
    grid = (E, H // hblk)
    return pl.pallas_call(
        _ffn_body,
        grid=grid,
        in_specs=[
            pl.BlockSpec((rows, C), lambda e, h: (e, 0)),
            pl.BlockSpec((1, C, hblk), lambda e, h: (e, 0, h)),
            pl.BlockSpec((1, 1, hblk), lambda e, h: (e, 0, h)),
            pl.BlockSpec((1, C, hblk), lambda e, h: (e, 0, h)),
            pl.BlockSpec((1, 1, hblk), lambda e, h: (e, 0, h)),
            pl.BlockSpec((1, hblk, C), lambda e, h: (e, h, 0)),
            pl.BlockSpec((1, 1, C), lambda e, h: (e, 0, 0)),
        ],
        out_specs=pl.BlockSpec((rows, C), lambda e, h: (e, 0)),
        out_shape=jax.ShapeDtypeStruct((n_rows, C), jnp.float32),
    )(ei, w1, b1, wg, bg, w2, b2)


def _make_gather(T, n_tok, C, chunk):
    n_chunks = n_tok // chunk
    cpb = T // chunk
    mesh = plsc.VectorSubcoreMesh(core_axis_name="c", subcore_axis_name="s")
    nw = mesh.num_cores * mesh.num_subcores
    per_w = n_chunks // nw

    n_jobs = per_w * _K

    @functools.partial(
        pl.kernel,
        out_type=[
            jax.ShapeDtypeStruct((n_tok, C), jnp.float32),
            jax.ShapeDtypeStruct((n_tok, C), jnp.float32),
        ],
        mesh=mesh,
        scratch_types=[
            pltpu.VMEM((2, chunk), jnp.int32),
            pltpu.VMEM((chunk, C), jnp.float32),
            pltpu.VMEM((chunk, C), jnp.float32),
            pltpu.SemaphoreType.DMA,
            pltpu.SemaphoreType.DMA,
            pltpu.SemaphoreType.DMA,
            pltpu.SemaphoreType.DMA,
        ],
    )
    def gather(eo_hbm, gcl_hbm, out0_hbm, out1_hbm,
               idx_v, rows0, rows1, g0, g1, s0, s1):
        wid = lax.axis_index("s") * mesh.num_cores + lax.axis_index("c")
        rows = (rows0, rows1)
        gsem = (g0, g1)
        ssem = (s0, s1)
        outs = (out0_hbm, out1_hbm)
        hg = [None, None]
        hs = [None, None]
        jobs = []
        for i in range(per_w):
            for k in range(_K):
                jobs.append((i, k))
        for j in range(n_jobs + 1):
            cur = j % 2
            if j < n_jobs:
                i, k = jobs[j]
                g = wid * per_w + i
                b = g // cpb
                goff = (b * _K * T + k * T + (g % cpb) * chunk).astype(jnp.int32)
                if hs[cur] is not None:
                    hs[cur].wait()
                pltpu.sync_copy(gcl_hbm.at[pl.ds(goff, chunk)], idx_v.at[cur])
                hg[cur] = pltpu.async_copy(eo_hbm.at[idx_v.at[cur]],
                                           rows[cur], gsem[cur])
            if j >= 1:
                prev = (j - 1) % 2
                pi, pk = jobs[j - 1]
                pbase = (wid * per_w + pi) * chunk
                hg[prev].wait()
                hs[prev] = pltpu.async_copy(rows[prev],
                                            outs[pk].at[pl.ds(pbase, chunk)],
                                            ssem[prev])
        for h in hs:
            if h is not None:
                h.wait()

    return gather


def _combine_body(g0_ref, g1_ref, p0_ref, p1_ref, y_ref):
    y_ref[...] = p0_ref[...] * g0_ref[...] + p1_ref[...] * g1_ref[...]


def _run_combine(g0, g1, probs_flat, T, blk):
    n, C = g0.shape
    cpb = T // blk
    return pl.pallas_call(
        _combine_body,
        grid=(n // blk,),
        in_specs=[
            pl.BlockSpec((blk, C), lambda i: (i, 0)),
            pl.BlockSpec((blk, C), lambda i: (i, 0)),
            pl.BlockSpec((blk, 1), lambda i: (i + (i // cpb) * cpb, 0)),
            pl.BlockSpec((blk, 1), lambda i: (i + (i // cpb) * cpb + cpb, 0)),
        ],
        out_specs=pl.BlockSpec((blk, C), lambda i: (i, 0)),
        out_shape=jax.ShapeDtypeStruct((n, C), jnp.float32),
    )(g0, g1, probs_flat, probs_flat)


def kernel(x, router_w, router_b, w_c_fc, b_c_fc, w_gate, b_gate, w_c_proj, b_c_proj):
    B, T, C = x.shape
    E = router_w.shape[1]
    cap = int(_LOAD * _K * T // E)
    n_rows = E * B * cap
    n_rows_pad = n_rows + 8
    dump_row = n_rows

    dst, gcl, probs = _run_router(x, router_w, router_b, cap, dump_row)

    ei = _make_dispatch(T, B * _K * T, n_rows_pad, C, 64)(x.reshape(-1, C),
                                                          dst.reshape(-1))

    hblk = 1536 if (w_c_fc.shape[2] % 1536 == 0) else w_c_fc.shape[2]
    eo = _run_ffn(ei, w_c_fc, b_c_fc, w_gate, b_gate, w_c_proj, b_c_proj,
                  n_rows, hblk)

    r0, r1 = _make_gather(T, B * T, C, 64)(eo, gcl.reshape(-1))

    y = _run_combine(r0, r1, probs.reshape(-1, 1), T, min(512, T))
    return y.reshape(B, T, C)
